# baseline (speedup 1.0000x reference)
import functools
import math

import jax
import jax.numpy as jnp
from jax import lax
from jax.experimental import pallas as pl
from jax.experimental.pallas import tpu as pltpu
from jax.experimental.pallas import tpu_sc as plsc

_LANES = 16


def _sc_router_body(num_tokens, num_experts, capacity, n_sub,
                    xT_hbm, rT_hbm, wT_hbm, used_hbm,
                    xloc, t1loc, t2loc, v1loc, v2loc,
                    sh_t1, sh_t2, sh_v1, sh_v2,
                    a_t1, a_t2, a_v1, a_v2, rrow, wrow, uvec):
    chunk = num_tokens // n_sub
    sid = lax.axis_index("s")
    base = sid * chunk

    pltpu.sync_copy(xT_hbm.at[:, pl.ds(base, chunk)], xloc)
    for g in range(chunk // _LANES):
        off = g * _LANES
        ps = [xloc[e, pl.ds(off, _LANES)] for e in range(num_experts)]
        m = ps[0]
        for e in range(1, num_experts):
            m = jnp.maximum(m, ps[e])
        es = [jnp.exp(p - m) for p in ps]
        s = es[0]
        for e in range(1, num_experts):
            s = s + es[e]
        inv = 1.0 / s
        probs = [t * inv for t in es]
        val1 = probs[0]
        idx1 = jnp.zeros((_LANES,), jnp.int32)
        for e in range(1, num_experts):
            gt = probs[e] > val1
            val1 = jnp.where(gt, probs[e], val1)
            idx1 = jnp.where(gt, jnp.full((_LANES,), e, jnp.int32), idx1)
        val2 = jnp.full((_LANES,), -jnp.inf, jnp.float32)
        idx2 = jnp.zeros((_LANES,), jnp.int32)
        for e in range(num_experts):
            ok = (idx1 != e) & (probs[e] > val2)
            val2 = jnp.where(ok, probs[e], val2)
            idx2 = jnp.where(ok, jnp.full((_LANES,), e, jnp.int32), idx2)
        t1loc[pl.ds(off, _LANES)] = idx1
        t2loc[pl.ds(off, _LANES)] = idx2
        v1loc[pl.ds(off, _LANES)] = val1
        v2loc[pl.ds(off, _LANES)] = val2
    pltpu.sync_copy(t1loc, sh_t1.at[pl.ds(base, chunk)])
    pltpu.sync_copy(t2loc, sh_t2.at[pl.ds(base, chunk)])
    pltpu.sync_copy(v1loc, sh_v1.at[pl.ds(base, chunk)])
    pltpu.sync_copy(v2loc, sh_v2.at[pl.ds(base, chunk)])
    plsc.subcore_barrier()

    @pl.when(sid < num_experts)
    def _phase2():
        e = sid
        pltpu.sync_copy(sh_t1, a_t1)
        pltpu.sync_copy(sh_t2, a_t2)
        pltpu.sync_copy(sh_v1, a_v1)
        pltpu.sync_copy(sh_v2, a_v2)
        n_grp = num_tokens // _LANES
        zero = jnp.zeros((), jnp.int32)

        def body_a(g, tot):
            t1v = a_t1[pl.ds(g * _LANES, _LANES)]
            return tot + jnp.sum((t1v == e).astype(jnp.int32))
        tot1 = lax.fori_loop(0, n_grp, body_a, zero)

        def body_b(g, carry):
            c1, c2 = carry
            o = g * _LANES
            t1v = a_t1[pl.ds(o, _LANES)]
            t2v = a_t2[pl.ds(o, _LANES)]
            v1v = a_v1[pl.ds(o, _LANES)]
            v2v = a_v2[pl.ds(o, _LANES)]
            m1 = t1v == e
            m2 = t2v == e
            s1 = plsc.cumsum(jnp.where(m1, 1, 0))
            s2 = plsc.cumsum(jnp.where(m2, 1, 0))
            r1 = c1 + s1 - 1
            r2 = c2 + s2 - 1 + tot1
            k1 = m1 & (r1 < capacity)
            k2 = m2 & (r2 < capacity)
            rr = jnp.where(k1, r1, jnp.where(k2, r2, jnp.full((_LANES,), -1, jnp.int32)))
            ww = jnp.where(k1, v1v, jnp.where(k2, v2v, jnp.zeros((_LANES,), jnp.float32)))
            rrow[pl.ds(o, _LANES)] = rr
            wrow[pl.ds(o, _LANES)] = ww
            c1 = c1 + jnp.sum(jnp.where(m1, 1, 0))
            c2 = c2 + jnp.sum(jnp.where(m2, 1, 0))
            return c1, c2
        c1f, c2f = lax.fori_loop(0, n_grp, body_b, (zero, zero))

        pltpu.sync_copy(rrow, rT_hbm.at[e])
        pltpu.sync_copy(wrow, wT_hbm.at[e])
        uvec[...] = jnp.full((_LANES,), jnp.minimum(tot1 + c2f, capacity),
                             jnp.int32)
        pltpu.sync_copy(uvec, used_hbm.at[e])


def _sc_router(xT, capacity):
    num_experts, num_tokens = xT.shape
    n_sub = 16
    mesh = plsc.VectorSubcoreMesh(
        core_axis_name="c", subcore_axis_name="s", num_cores=1,
        num_subcores=n_sub)
    chunk = num_tokens // n_sub
    body = functools.partial(_sc_router_body, num_tokens, num_experts,
                             capacity, n_sub)
    return pl.kernel(
        body,
        out_type=(
            jax.ShapeDtypeStruct((num_experts, num_tokens), jnp.int32),
            jax.ShapeDtypeStruct((num_experts, num_tokens), jnp.float32),
            jax.ShapeDtypeStruct((num_experts, _LANES), jnp.int32),
        ),
        mesh=mesh,
        compiler_params=pltpu.CompilerParams(needs_layout_passes=False),
        scratch_types=[
            pltpu.VMEM((num_experts, chunk), jnp.float32),
            pltpu.VMEM((chunk,), jnp.int32),
            pltpu.VMEM((chunk,), jnp.int32),
            pltpu.VMEM((chunk,), jnp.float32),
            pltpu.VMEM((chunk,), jnp.float32),
            pltpu.VMEM_SHARED((num_tokens,), jnp.int32),
            pltpu.VMEM_SHARED((num_tokens,), jnp.int32),
            pltpu.VMEM_SHARED((num_tokens,), jnp.float32),
            pltpu.VMEM_SHARED((num_tokens,), jnp.float32),
            pltpu.VMEM((num_tokens,), jnp.int32),
            pltpu.VMEM((num_tokens,), jnp.int32),
            pltpu.VMEM((num_tokens,), jnp.float32),
            pltpu.VMEM((num_tokens,), jnp.float32),
            pltpu.VMEM((num_tokens,), jnp.int32),
            pltpu.VMEM((num_tokens,), jnp.float32),
            pltpu.VMEM((_LANES,), jnp.int32),
        ],
    )(xT)


def _tc_materialize_body(capacity, blk, num_experts, r_ref, w_ref, cb_ref, mask_ref):
    r = r_ref[...][:, :, None]
    w = w_ref[...][:, :, None]
    iota = lax.broadcasted_iota(jnp.int32, (blk, num_experts, capacity), 2)
    eq = iota == r
    cb_ref[...] = jnp.where(eq, w, jnp.zeros_like(w))
    mask_ref[...] = eq & (w > 0)


def _tc_materialize(r, w, capacity):
    num_tokens, num_experts = r.shape
    blk = 256
    body = functools.partial(_tc_materialize_body, capacity, blk, num_experts)
    return pl.pallas_call(
        body,
        grid=(num_tokens // blk,),
        in_specs=[
            pl.BlockSpec((blk, num_experts), lambda i: (i, 0)),
            pl.BlockSpec((blk, num_experts), lambda i: (i, 0)),
        ],
        out_specs=[
            pl.BlockSpec((blk, num_experts, capacity), lambda i: (i, 0, 0)),
            pl.BlockSpec((blk, num_experts, capacity), lambda i: (i, 0, 0)),
        ],
        out_shape=[
            jax.ShapeDtypeStruct((num_tokens, num_experts, capacity), jnp.float32),
            jax.ShapeDtypeStruct((num_tokens, num_experts, capacity), jnp.bool_),
        ],
    )(r, w)


def kernel(inputs):
    num_tokens, num_experts = inputs.shape
    capacity = math.floor(2 * 1.25 * num_tokens / num_experts)
    capacity += capacity % 2
    capacity = max(capacity, 4)

    xT = inputs.T
    rT, wT, used_m = _sc_router(xT, capacity)
    used_capacity = used_m[:, 0]
    cb_weight, sec_mask = _tc_materialize(rT.T, wT.T, capacity)
    return used_capacity, cb_weight, sec_mask
